# baseline (speedup 1.0000x reference)
import jax, jax.numpy as jnp
from jax.experimental import pallas as pl

N, E, D = 10000, 320000, 128

def _tc_body(s_ref, m_ref, c_ref, wt_ref, b_ref, o_ref):
    s = s_ref[...]
    mx = m_ref[...]
    c = c_ref[...]
    mean = s / jnp.maximum(c, 1.0)
    mxf = jnp.where(c > 0.0, mx, 0.0)
    acc = jnp.dot(s, wt_ref[0:D, :], preferred_element_type=jnp.float32)
    acc = acc + jnp.dot(mean, wt_ref[D:2*D, :], preferred_element_type=jnp.float32)
    acc = acc + jnp.dot(mxf, wt_ref[2*D:3*D, :], preferred_element_type=jnp.float32)
    o_ref[...] = acc + b_ref[...]

def kernel(x, edge_index, W, b):
    src, dst = edge_index[0], edge_index[1]
    msgs = jnp.take(x, src, axis=0)
    s = jax.ops.segment_sum(msgs, dst, num_segments=N)
    deg = jax.ops.segment_sum(jnp.ones((E,), x.dtype), dst, num_segments=N)
    mx = jax.ops.segment_max(msgs, dst, num_segments=N)
    mx = jnp.where(deg[:, None] > 0, mx, 0.0)
    return pl.pallas_call(
        _tc_body,
        grid=(5,),
        in_specs=[
            pl.BlockSpec((2000, D), lambda i: (i, 0)),
            pl.BlockSpec((2000, D), lambda i: (i, 0)),
            pl.BlockSpec((2000, 1), lambda i: (i, 0)),
            pl.BlockSpec((3*D, D), lambda i: (0, 0)),
            pl.BlockSpec((1, D), lambda i: (0, 0)),
        ],
        out_specs=pl.BlockSpec((2000, D), lambda i: (i, 0)),
        out_shape=jax.ShapeDtypeStruct((N, D), jnp.float32),
    )(s, mx, deg[:, None], W.T, b.reshape(1, D))


# speedup vs baseline: 2.2358x; 2.2358x over previous
import dataclasses
import functools

import jax
import jax.numpy as jnp
from jax import lax
from jax.experimental import pallas as pl
from jax.experimental.pallas import tpu as pltpu
from jax.experimental.pallas import tpu_sc as plsc

N = 10000
E = 320000
D = 128
NF = D // 16
NT = 32
R = 320
NPAD = NT * R
CH = 1600
NCH = E // CH
NG = CH // 16
GB = 128
FL = 1856
NEG = -3.0e38

_mesh = plsc.VectorSubcoreMesh(core_axis_name="c", subcore_axis_name="s")

_sc_params = pltpu.CompilerParams(
    needs_layout_passes=False, use_tc_tiling_on_sc=False)


def _sc_body(x_hbm, src_hbm, dst_hbm, sum_hbm, max_hbm, cnt_hbm,
             acc_sum, acc_max, acc_cnt, dstb, srcb, fl_src, fl_dloc,
             gbuf, gidx, sem):
    wid = lax.axis_index("s") * 2 + lax.axis_index("c")
    lo = wid * R
    hi = lo + R

    zeros_f = jnp.zeros((16,), jnp.float32)
    neg_f = jnp.full((16,), NEG, jnp.float32)
    zeros_i = jnp.zeros((16,), jnp.int32)

    def init_row(r, carry):
        for f in range(NF):
            sl = pl.ds(f * 16, 16)
            acc_sum[r, sl] = zeros_f
            acc_max[r, sl] = neg_f
        acc_cnt[r, :] = zeros_f
        return carry
    lax.fori_loop(0, R, init_row, 0)

    def init_fl(g, carry):
        fl_src[pl.ds(pl.multiple_of(g * 16, 16), 16)] = zeros_i
        return carry
    lax.fori_loop(0, FL // 16, init_fl, 0)

    lanes = lax.iota(jnp.int32, 16)

    def flush(p, nvalid):
        p = pl.multiple_of(p, 16)
        for g in range(GB // 16):
            gidx[pl.ds(g * 16, 16)] = (
                fl_src[pl.ds(pl.multiple_of(p + g * 16, 16), 16)])
        pltpu.sync_copy(x_hbm.at[gidx], gbuf)

        def edge_body(e, carry):
            g16 = pl.multiple_of((e // 16) * 16, 16)
            grp = fl_dloc[pl.ds(pl.multiple_of(p + g16, 16), 16)]
            dl = jnp.sum(jnp.where(lanes == e - g16, grp, 0))
            dl = jnp.clip(dl, 0, R - 1)
            for f in range(NF):
                sl = pl.ds(f * 16, 16)
                row = gbuf[e, sl]
                acc_sum[dl, sl] = acc_sum[dl, sl] + row
                acc_max[dl, sl] = jnp.maximum(acc_max[dl, sl], row)
            acc_cnt[dl, :] = acc_cnt[dl, :] + 1.0
            return carry
        lax.fori_loop(0, nvalid, edge_body, 0)

    def drain(off):
        def flush_cond(p):
            return p + GB <= off

        def flush_batch(p):
            flush(p, GB)
            return p + GB
        p = lax.while_loop(flush_cond, flush_batch, jnp.int32(0))

        def shift_body(k, carry):
            ksl = pl.ds(pl.multiple_of(k * 16, 16), 16)
            psl = pl.ds(pl.multiple_of(p + k * 16, 16), 16)
            fl_src[ksl] = fl_src[psl]
            fl_dloc[ksl] = fl_dloc[psl]
            return carry
        lax.fori_loop(0, GB // 16, shift_body, 0)
        return off - p

    def chunk_body(c, off):
        base = pl.multiple_of(c * CH, 16)
        pltpu.sync_copy(dst_hbm.at[pl.ds(base, CH)], dstb)
        pltpu.sync_copy(src_hbm.at[pl.ds(base, CH)], srcb)

        off = drain(off)

        def group_body(g, off):
            gb = pl.multiple_of(g * 16, 16)
            sl = pl.ds(gb, 16)
            d = dstb[sl]
            m = (d >= lo) & (d < hi)
            s = srcb[sl]
            mi = m.astype(jnp.int32)
            pos = off + plsc.cumsum(mi) - 1
            plsc.store_scatter(fl_src, [pos], s, mask=m)
            plsc.store_scatter(fl_dloc, [pos], d - lo, mask=m)
            return off + jnp.sum(mi)
        return lax.fori_loop(0, NG, group_body, off)

    off = lax.fori_loop(0, NCH, chunk_body, jnp.int32(0))

    pltpu.sync_copy(dst_hbm.at[pl.ds(0, CH)], dstb)
    off = drain(off)
    flush(0, off)

    pltpu.sync_copy(acc_sum, sum_hbm.at[pl.ds(lo, R)])
    pltpu.sync_copy(acc_max, max_hbm.at[pl.ds(lo, R)])
    pltpu.sync_copy(acc_cnt, cnt_hbm.at[pl.ds(lo, R)])


_sc_aggregate = functools.partial(
    pl.kernel,
    out_type=(
        jax.ShapeDtypeStruct((NPAD, D), jnp.float32),
        jax.ShapeDtypeStruct((NPAD, D), jnp.float32),
        jax.ShapeDtypeStruct((NPAD, 16), jnp.float32),
    ),
    mesh=_mesh,
    scratch_types=[
        pltpu.VMEM((R, D), jnp.float32),
        pltpu.VMEM((R, D), jnp.float32),
        pltpu.VMEM((R, 16), jnp.float32),
        pltpu.VMEM((CH,), jnp.int32),
        pltpu.VMEM((CH,), jnp.int32),
        pltpu.VMEM((FL,), jnp.int32),
        pltpu.VMEM((FL,), jnp.int32),
        pltpu.VMEM((GB, D), jnp.float32),
        pltpu.VMEM((GB,), jnp.int32),
        pltpu.SemaphoreType.DMA,
    ],
    compiler_params=_sc_params,
)(_sc_body)


BLK = NPAD // 4


def _tc_body(sum_ref, max_ref, cnt_ref, wt_ref, b_ref, o_ref):
    s = sum_ref[...]
    mx = max_ref[...]
    c = cnt_ref[:, 0:1]
    mean = s / jnp.maximum(c, 1.0)
    mxf = jnp.where(c > 0.0, mx, 0.0)
    acc = jnp.dot(s, wt_ref[0:D, :], preferred_element_type=jnp.float32)
    acc = acc + jnp.dot(mean, wt_ref[D:2 * D, :],
                        preferred_element_type=jnp.float32)
    acc = acc + jnp.dot(mxf, wt_ref[2 * D:3 * D, :],
                        preferred_element_type=jnp.float32)
    o_ref[...] = acc + b_ref[...]


def _tc_finish(sums, maxs, cnts, wt, b2):
    return pl.pallas_call(
        _tc_body,
        grid=(NPAD // BLK,),
        in_specs=[
            pl.BlockSpec((BLK, D), lambda i: (i, 0)),
            pl.BlockSpec((BLK, D), lambda i: (i, 0)),
            pl.BlockSpec((BLK, 16), lambda i: (i, 0)),
            pl.BlockSpec((3 * D, D), lambda i: (0, 0)),
            pl.BlockSpec((1, D), lambda i: (0, 0)),
        ],
        out_specs=pl.BlockSpec((BLK, D), lambda i: (i, 0)),
        out_shape=jax.ShapeDtypeStruct((NPAD, D), jnp.float32),
    )(sums, maxs, cnts, wt, b2)


def kernel(x, edge_index, W, b):
    src = edge_index[0]
    dst = edge_index[1]
    sums, maxs, cnts = _sc_aggregate(x, src, dst)
    out = _tc_finish(sums, maxs, cnts, W.T, b.reshape(1, D))
    return out[:N]


# speedup vs baseline: 2.4033x; 1.0749x over previous
import dataclasses
import functools

import jax
import jax.numpy as jnp
from jax import lax
from jax.experimental import pallas as pl
from jax.experimental.pallas import tpu as pltpu
from jax.experimental.pallas import tpu_sc as plsc

N = 10000
E = 320000
D = 128
NF = D // 16
NT = 32
R = 320
NPAD = NT * R
CH = 1600
NCH = E // CH
NG = CH // 16
GB = 128
FL = 1856
NEG = -3.0e38

_mesh = plsc.VectorSubcoreMesh(core_axis_name="c", subcore_axis_name="s")

_sc_params = pltpu.CompilerParams(
    needs_layout_passes=False, use_tc_tiling_on_sc=False)


def _sc_body(x_hbm, src_hbm, dst_hbm, sum_hbm, max_hbm, cnt_hbm,
             acc_sum, acc_max, acc_cnt, dstb, srcb, fl_src, fl_dloc,
             gbuf, gidx, sem):
    wid = lax.axis_index("s") * 2 + lax.axis_index("c")
    lo = wid * R
    hi = lo + R

    zeros_f = jnp.zeros((16,), jnp.float32)
    neg_f = jnp.full((16,), NEG, jnp.float32)
    zeros_i = jnp.zeros((16,), jnp.int32)

    def init_row(r, carry):
        for f in range(NF):
            sl = pl.ds(f * 16, 16)
            acc_sum[r, sl] = zeros_f
            acc_max[r, sl] = neg_f
        acc_cnt[r, :] = zeros_f
        return carry
    lax.fori_loop(0, R, init_row, 0)

    def init_fl(g, carry):
        fl_src[pl.ds(pl.multiple_of(g * 16, 16), 16)] = zeros_i
        return carry
    lax.fori_loop(0, FL // 16, init_fl, 0)

    lanes = lax.iota(jnp.int32, 16)

    def flush(p, nvalid):
        p = pl.multiple_of(p, 16)
        for g in range(GB // 16):
            gidx[pl.ds(g * 16, 16)] = (
                fl_src[pl.ds(pl.multiple_of(p + g * 16, 16), 16)])
        pltpu.sync_copy(x_hbm.at[gidx], gbuf)

        def edge_body(e, carry):
            g16 = pl.multiple_of((e // 16) * 16, 16)
            grp = fl_dloc[pl.ds(pl.multiple_of(p + g16, 16), 16)]
            dl = jnp.sum(jnp.where(lanes == e - g16, grp, 0))
            dl = jnp.clip(dl, 0, R - 1)
            for f in range(NF):
                sl = pl.ds(f * 16, 16)
                row = gbuf[e, sl]
                acc_sum[dl, sl] = acc_sum[dl, sl] + row
                acc_max[dl, sl] = jnp.maximum(acc_max[dl, sl], row)
            acc_cnt[dl, :] = acc_cnt[dl, :] + 1.0
            return carry
        lax.fori_loop(0, nvalid, edge_body, 0)

    def drain(off):
        def flush_cond(p):
            return p + GB <= off

        def flush_batch(p):
            flush(p, GB)
            return p + GB
        p = lax.while_loop(flush_cond, flush_batch, jnp.int32(0))

        def shift_body(k, carry):
            ksl = pl.ds(pl.multiple_of(k * 16, 16), 16)
            psl = pl.ds(pl.multiple_of(p + k * 16, 16), 16)
            fl_src[ksl] = fl_src[psl]
            fl_dloc[ksl] = fl_dloc[psl]
            return carry
        lax.fori_loop(0, GB // 16, shift_body, 0)
        return off - p

    def chunk_body(c, off):
        base = pl.multiple_of(c * CH, 16)
        cpd = pltpu.async_copy(dst_hbm.at[pl.ds(base, CH)], dstb, sem)
        cps = pltpu.async_copy(src_hbm.at[pl.ds(base, CH)], srcb, sem)
        cpd.wait()
        cps.wait()

        off = drain(off)

        offv = jnp.full((16,), 0, jnp.int32) + off

        def group_body(g, offv):
            gb = pl.multiple_of(g * 16, 16)
            sl = pl.ds(gb, 16)
            d = dstb[sl]
            m = (d >= lo) & (d < hi)
            s = srcb[sl]
            mi = m.astype(jnp.int32)
            pos = offv + plsc.cumsum(mi) - 1
            plsc.store_scatter(fl_src, [pos], s, mask=m)
            plsc.store_scatter(fl_dloc, [pos], d - lo, mask=m)
            return offv + plsc.all_reduce_population_count(m)
        offv = lax.fori_loop(0, NG, group_body, offv)
        return jnp.max(offv)

    off = lax.fori_loop(0, NCH, chunk_body, jnp.int32(0))

    pltpu.sync_copy(dst_hbm.at[pl.ds(0, CH)], dstb)
    off = drain(off)
    flush(0, off)

    pltpu.sync_copy(acc_sum, sum_hbm.at[pl.ds(lo, R)])
    pltpu.sync_copy(acc_max, max_hbm.at[pl.ds(lo, R)])
    pltpu.sync_copy(acc_cnt, cnt_hbm.at[pl.ds(lo, R)])


_sc_aggregate = functools.partial(
    pl.kernel,
    out_type=(
        jax.ShapeDtypeStruct((NPAD, D), jnp.float32),
        jax.ShapeDtypeStruct((NPAD, D), jnp.float32),
        jax.ShapeDtypeStruct((NPAD, 16), jnp.float32),
    ),
    mesh=_mesh,
    scratch_types=[
        pltpu.VMEM((R, D), jnp.float32),
        pltpu.VMEM((R, D), jnp.float32),
        pltpu.VMEM((R, 16), jnp.float32),
        pltpu.VMEM((CH,), jnp.int32),
        pltpu.VMEM((CH,), jnp.int32),
        pltpu.VMEM((FL,), jnp.int32),
        pltpu.VMEM((FL,), jnp.int32),
        pltpu.VMEM((GB, D), jnp.float32),
        pltpu.VMEM((GB,), jnp.int32),
        pltpu.SemaphoreType.DMA,
    ],
    compiler_params=_sc_params,
)(_sc_body)


BLK = NPAD // 4


def _tc_body(sum_ref, max_ref, cnt_ref, wt_ref, b_ref, o_ref):
    s = sum_ref[...]
    mx = max_ref[...]
    c = cnt_ref[:, 0:1]
    mean = s / jnp.maximum(c, 1.0)
    mxf = jnp.where(c > 0.0, mx, 0.0)
    acc = jnp.dot(s, wt_ref[0:D, :], preferred_element_type=jnp.float32)
    acc = acc + jnp.dot(mean, wt_ref[D:2 * D, :],
                        preferred_element_type=jnp.float32)
    acc = acc + jnp.dot(mxf, wt_ref[2 * D:3 * D, :],
                        preferred_element_type=jnp.float32)
    o_ref[...] = acc + b_ref[...]


def _tc_finish(sums, maxs, cnts, wt, b2):
    return pl.pallas_call(
        _tc_body,
        grid=(NPAD // BLK,),
        in_specs=[
            pl.BlockSpec((BLK, D), lambda i: (i, 0)),
            pl.BlockSpec((BLK, D), lambda i: (i, 0)),
            pl.BlockSpec((BLK, 16), lambda i: (i, 0)),
            pl.BlockSpec((3 * D, D), lambda i: (0, 0)),
            pl.BlockSpec((1, D), lambda i: (0, 0)),
        ],
        out_specs=pl.BlockSpec((BLK, D), lambda i: (i, 0)),
        out_shape=jax.ShapeDtypeStruct((NPAD, D), jnp.float32),
    )(sums, maxs, cnts, wt, b2)


def kernel(x, edge_index, W, b):
    src = edge_index[0]
    dst = edge_index[1]
    sums, maxs, cnts = _sc_aggregate(x, src, dst)
    out = _tc_finish(sums, maxs, cnts, W.T, b.reshape(1, D))
    return out[:N]


# speedup vs baseline: 2.4882x; 1.0353x over previous
import dataclasses
import functools

import jax
import jax.numpy as jnp
from jax import lax
from jax.experimental import pallas as pl
from jax.experimental.pallas import tpu as pltpu
from jax.experimental.pallas import tpu_sc as plsc

N = 10000
E = 320000
D = 128
NF = D // 16
NT = 32
R = 320
NPAD = NT * R
CH = 1600
NCH = E // CH
NG = CH // 16
GB = 128
FL = 1856
NEG = -3.0e38

_mesh = plsc.VectorSubcoreMesh(core_axis_name="c", subcore_axis_name="s")

_sc_params = pltpu.CompilerParams(
    needs_layout_passes=False, use_tc_tiling_on_sc=False)


def _sc_body(x_hbm, src_hbm, dst_hbm, sum_hbm, max_hbm, cnt_hbm,
             acc_sum, acc_max, acc_cnt, dstb, srcb, fl_src, fl_dloc,
             gbuf, gidx, sem):
    wid = lax.axis_index("s") * 2 + lax.axis_index("c")
    lo = wid * R
    hi = lo + R

    zeros_f = jnp.zeros((16,), jnp.float32)
    neg_f = jnp.full((16,), NEG, jnp.float32)
    zeros_i = jnp.zeros((16,), jnp.int32)

    def init_row(r, carry):
        for f in range(NF):
            sl = pl.ds(f * 16, 16)
            acc_sum[r, sl] = zeros_f
            acc_max[r, sl] = neg_f
        acc_cnt[r, :] = zeros_f
        return carry
    lax.fori_loop(0, R, init_row, 0)

    def init_fl(g, carry):
        fl_src[pl.ds(pl.multiple_of(g * 16, 16), 16)] = zeros_i
        return carry
    lax.fori_loop(0, FL // 16, init_fl, 0)

    lanes = lax.iota(jnp.int32, 16)

    ones_f = jnp.full((16,), 1.0, jnp.float32)
    hot = [jnp.asarray(jnp.arange(16) == j) for j in range(16)]

    def _accumulate(e, dl):
        for f in range(NF):
            sl = pl.ds(f * 16, 16)
            row = gbuf[e, sl]
            plsc.addupdate(acc_sum.at[dl, sl], row)
            acc_max[dl, sl] = jnp.maximum(acc_max[dl, sl], row)
        plsc.addupdate(acc_cnt.at[dl, :], ones_f)

    def _stage_gather(p):
        for g in range(GB // 16):
            gidx[pl.ds(g * 16, 16)] = (
                fl_src[pl.ds(pl.multiple_of(p + g * 16, 16), 16)])
        pltpu.sync_copy(x_hbm.at[gidx], gbuf)

    def flush_full(p):
        p = pl.multiple_of(p, 16)
        _stage_gather(p)

        def group_body(g, carry):
            g16 = pl.multiple_of(g * 16, 16)
            grp = fl_dloc[pl.ds(pl.multiple_of(p + g16, 16), 16)]
            for j in range(16):
                dl = jnp.sum(jnp.where(hot[j], grp, 0))
                _accumulate(g16 + j, dl)
            return carry
        lax.fori_loop(0, GB // 16, group_body, 0)

    def flush(p, nvalid):
        p = pl.multiple_of(p, 16)
        _stage_gather(p)

        def edge_body(e, carry):
            g16 = pl.multiple_of((e // 16) * 16, 16)
            grp = fl_dloc[pl.ds(pl.multiple_of(p + g16, 16), 16)]
            dl = jnp.sum(jnp.where(lanes == e - g16, grp, 0))
            _accumulate(e, dl)
            return carry
        lax.fori_loop(0, nvalid, edge_body, 0)

    def drain(off):
        def flush_cond(p):
            return p + GB <= off

        def flush_batch(p):
            flush_full(p)
            return p + GB
        p = lax.while_loop(flush_cond, flush_batch, jnp.int32(0))

        def shift_body(k, carry):
            ksl = pl.ds(pl.multiple_of(k * 16, 16), 16)
            psl = pl.ds(pl.multiple_of(p + k * 16, 16), 16)
            fl_src[ksl] = fl_src[psl]
            fl_dloc[ksl] = fl_dloc[psl]
            return carry
        lax.fori_loop(0, GB // 16, shift_body, 0)
        return off - p

    def chunk_body(c, off):
        base = pl.multiple_of(c * CH, 16)
        cpd = pltpu.async_copy(dst_hbm.at[pl.ds(base, CH)], dstb, sem)
        cps = pltpu.async_copy(src_hbm.at[pl.ds(base, CH)], srcb, sem)
        cpd.wait()
        cps.wait()

        off = drain(off)

        offv = jnp.full((16,), 0, jnp.int32) + off

        def group_body(g, offv):
            gb = pl.multiple_of(g * 16, 16)
            sl = pl.ds(gb, 16)
            d = dstb[sl]
            m = (d >= lo) & (d < hi)
            s = srcb[sl]
            mi = m.astype(jnp.int32)
            pos = offv + plsc.cumsum(mi) - 1
            plsc.store_scatter(fl_src, [pos], s, mask=m)
            plsc.store_scatter(fl_dloc, [pos], d - lo, mask=m)
            return offv + plsc.all_reduce_population_count(m)
        offv = lax.fori_loop(0, NG, group_body, offv)
        return jnp.max(offv)

    off = lax.fori_loop(0, NCH, chunk_body, jnp.int32(0))

    pltpu.sync_copy(dst_hbm.at[pl.ds(0, CH)], dstb)
    off = drain(off)
    flush(0, off)

    pltpu.sync_copy(acc_sum, sum_hbm.at[pl.ds(lo, R)])
    pltpu.sync_copy(acc_max, max_hbm.at[pl.ds(lo, R)])
    pltpu.sync_copy(acc_cnt, cnt_hbm.at[pl.ds(lo, R)])


_sc_aggregate = functools.partial(
    pl.kernel,
    out_type=(
        jax.ShapeDtypeStruct((NPAD, D), jnp.float32),
        jax.ShapeDtypeStruct((NPAD, D), jnp.float32),
        jax.ShapeDtypeStruct((NPAD, 16), jnp.float32),
    ),
    mesh=_mesh,
    scratch_types=[
        pltpu.VMEM((R, D), jnp.float32),
        pltpu.VMEM((R, D), jnp.float32),
        pltpu.VMEM((R, 16), jnp.float32),
        pltpu.VMEM((CH,), jnp.int32),
        pltpu.VMEM((CH,), jnp.int32),
        pltpu.VMEM((FL,), jnp.int32),
        pltpu.VMEM((FL,), jnp.int32),
        pltpu.VMEM((GB, D), jnp.float32),
        pltpu.VMEM((GB,), jnp.int32),
        pltpu.SemaphoreType.DMA,
    ],
    compiler_params=_sc_params,
)(_sc_body)


BLK = NPAD // 4


def _tc_body(sum_ref, max_ref, cnt_ref, wt_ref, b_ref, o_ref):
    s = sum_ref[...]
    mx = max_ref[...]
    c = cnt_ref[:, 0:1]
    mean = s / jnp.maximum(c, 1.0)
    mxf = jnp.where(c > 0.0, mx, 0.0)
    acc = jnp.dot(s, wt_ref[0:D, :], preferred_element_type=jnp.float32)
    acc = acc + jnp.dot(mean, wt_ref[D:2 * D, :],
                        preferred_element_type=jnp.float32)
    acc = acc + jnp.dot(mxf, wt_ref[2 * D:3 * D, :],
                        preferred_element_type=jnp.float32)
    o_ref[...] = acc + b_ref[...]


def _tc_finish(sums, maxs, cnts, wt, b2):
    return pl.pallas_call(
        _tc_body,
        grid=(NPAD // BLK,),
        in_specs=[
            pl.BlockSpec((BLK, D), lambda i: (i, 0)),
            pl.BlockSpec((BLK, D), lambda i: (i, 0)),
            pl.BlockSpec((BLK, 16), lambda i: (i, 0)),
            pl.BlockSpec((3 * D, D), lambda i: (0, 0)),
            pl.BlockSpec((1, D), lambda i: (0, 0)),
        ],
        out_specs=pl.BlockSpec((BLK, D), lambda i: (i, 0)),
        out_shape=jax.ShapeDtypeStruct((NPAD, D), jnp.float32),
    )(sums, maxs, cnts, wt, b2)


def kernel(x, edge_index, W, b):
    src = edge_index[0]
    dst = edge_index[1]
    sums, maxs, cnts = _sc_aggregate(x, src, dst)
    out = _tc_finish(sums, maxs, cnts, W.T, b.reshape(1, D))
    return out[:N]
